# baseline (speedup 1.0000x reference)
import jax
import jax.numpy as jnp
from jax.experimental import pallas as pl

N = 10000
E = 160000
DIN = 256
DH = 256
DOUT = 128
NB = 64


def _fc_body(p_ref, w_ref, b_ref, o_ref):
    o_ref[...] = p_ref[...] @ w_ref[...].T + b_ref[...]


def _gat(x, src, dst, W, a_s, a_d, b):
    h = x @ W.T
    e = jnp.sum(h * a_s, axis=-1)[src] + jnp.sum(h * a_d, axis=-1)[dst]
    e = jax.nn.leaky_relu(e, 0.2)
    m = jax.ops.segment_max(e, dst, num_segments=N)
    m = jnp.where(jnp.isfinite(m), m, 0.0)
    p = jnp.exp(e - m[dst])
    s = jax.ops.segment_sum(p, dst, num_segments=N)
    alpha = p / (s[dst] + 1e-16)
    return jax.ops.segment_sum(h[src] * alpha[:, None], dst, num_segments=N) + b


def kernel(x, edge_index, batch, W1, att_src1, att_dst1, b1, W2, att_src2, att_dst2, b2, W3, att_src3, att_dst3, b3, Wf, bf):
    loop = jnp.arange(N, dtype=edge_index.dtype)
    src = jnp.concatenate([edge_index[0], loop])
    dst = jnp.concatenate([edge_index[1], loop])
    h = jax.nn.relu(_gat(x, src, dst, W1, att_src1, att_dst1, b1))
    h = jax.nn.relu(_gat(h, src, dst, W2, att_src2, att_dst2, b2))
    h = jax.nn.relu(_gat(h, src, dst, W3, att_src3, att_dst3, b3))
    cnt = jax.ops.segment_sum(jnp.ones((N,), jnp.float32), batch, num_segments=NB)
    pooled = jax.ops.segment_sum(h, batch, num_segments=NB) / jnp.maximum(cnt, 1.0)[:, None]
    return pl.pallas_call(
        _fc_body,
        out_shape=jax.ShapeDtypeStruct((NB, DOUT), jnp.float32),
    )(pooled, Wf, bf)


# speedup vs baseline: 1.7480x; 1.7480x over previous
import functools

import jax
import jax.numpy as jnp
from jax import lax
from jax.experimental import pallas as pl
from jax.experimental.pallas import tpu as pltpu
from jax.experimental.pallas import tpu_sc as plsc

N = 10000
E = 160000
DIN = 256
DH = 256
DOUT = 128
NB = 64
HD = 128

E_TOT = E + N
NBLK = 96
CE = 128
T_E = NBLK * CE
E_PAD = 16 * T_E
EROWS = E_PAD // CE
SROWS = 80
NP = 10240
PASS_R = 5120
NPASS = 2
ACC_R = PASS_R + 128
NEG = -1e30

_mesh = plsc.VectorSubcoreMesh(core_axis_name="c", subcore_axis_name="s")
_scp = pltpu.CompilerParams(use_tc_tiling_on_sc=False,
                            needs_layout_passes=False)


def _tc_in_body(x_ref, w_ref, as_ref, ad_ref, hh_ref, sc_ref):
    h = lax.dot_general(x_ref[...], w_ref[...], (((1,), (1,)), ((), ())),
                        preferred_element_type=jnp.float32)
    hh_ref[0] = h[:, :HD]
    hh_ref[1] = h[:, HD:]
    s0 = jnp.sum(h * as_ref[...], axis=1, keepdims=True)
    s1 = jnp.sum(h * ad_ref[...], axis=1, keepdims=True)
    sc_ref[...] = jnp.concatenate([s0, s1], axis=1)


_BM = 1000


def _tc_in(x, W, a_s, a_d):
    return pl.pallas_call(
        _tc_in_body,
        grid=(N // _BM,),
        in_specs=[
            pl.BlockSpec((_BM, DIN), lambda i: (i, 0)),
            pl.BlockSpec((DH, DIN), lambda i: (0, 0)),
            pl.BlockSpec((1, DH), lambda i: (0, 0)),
            pl.BlockSpec((1, DH), lambda i: (0, 0)),
        ],
        out_specs=[
            pl.BlockSpec((2, _BM, HD), lambda i: (0, i, 0)),
            pl.BlockSpec((_BM, 2), lambda i: (i, 0)),
        ],
        out_shape=[
            jax.ShapeDtypeStruct((2, NP, HD), jnp.float32),
            jax.ShapeDtypeStruct((NP, 2), jnp.float32),
        ],
    )(x, W, a_s, a_d)


def _tc_mid_body(a0_ref, a1_ref, b_ref, w_ref, as_ref, ad_ref,
                 hh_ref, sc_ref):
    x0 = jnp.maximum(a0_ref[...] + b_ref[:, :HD], 0.0)
    x1 = jnp.maximum(a1_ref[...] + b_ref[:, HD:], 0.0)
    h = (lax.dot_general(x0, w_ref[:, :HD], (((1,), (1,)), ((), ())),
                         preferred_element_type=jnp.float32)
         + lax.dot_general(x1, w_ref[:, HD:], (((1,), (1,)), ((), ())),
                           preferred_element_type=jnp.float32))
    hh_ref[0] = h[:, :HD]
    hh_ref[1] = h[:, HD:]
    s0 = jnp.sum(h * as_ref[...], axis=1, keepdims=True)
    s1 = jnp.sum(h * ad_ref[...], axis=1, keepdims=True)
    sc_ref[...] = jnp.concatenate([s0, s1], axis=1)


_BM2 = 1024


def _tc_mid(agg, b, W, a_s, a_d):
    nb2 = NP // _BM2
    return pl.pallas_call(
        _tc_mid_body,
        grid=(nb2,),
        in_specs=[
            pl.BlockSpec((_BM2, HD), lambda i: (i, 0)),
            pl.BlockSpec((_BM2, HD), lambda i, nb2=nb2: (i + nb2, 0)),
            pl.BlockSpec((1, DH), lambda i: (0, 0)),
            pl.BlockSpec((DH, DH), lambda i: (0, 0)),
            pl.BlockSpec((1, DH), lambda i: (0, 0)),
            pl.BlockSpec((1, DH), lambda i: (0, 0)),
        ],
        out_specs=[
            pl.BlockSpec((2, _BM2, HD), lambda i: (0, i, 0)),
            pl.BlockSpec((_BM2, 2), lambda i: (i, 0)),
        ],
        out_shape=[
            jax.ShapeDtypeStruct((2, NP, HD), jnp.float32),
            jax.ShapeDtypeStruct((NP, 2), jnp.float32),
        ],
    )(agg, agg, b, W, a_s, a_d)


_BMF = 512


def _tc_fin_body(a0_ref, a1_ref, b_ref, bt_ref, wf_ref, bf_ref, o_ref,
                 acc_ref, cnt_ref):
    i = pl.program_id(0)

    @pl.when(i == 0)
    def _():
        acc_ref[...] = jnp.zeros_like(acc_ref)
        cnt_ref[...] = jnp.zeros_like(cnt_ref)

    x0 = jnp.maximum(a0_ref[...] + b_ref[:, :HD], 0.0)
    x1 = jnp.maximum(a1_ref[...] + b_ref[:, HD:], 0.0)
    bt = bt_ref[0]
    oh = (lax.broadcasted_iota(jnp.int32, (NB, _BMF), 0) == bt
          ).astype(jnp.float32)
    acc_ref[...] += jnp.concatenate(
        [lax.dot_general(oh, x0, (((1,), (0,)), ((), ())),
                         preferred_element_type=jnp.float32),
         lax.dot_general(oh, x1, (((1,), (0,)), ((), ())),
                         preferred_element_type=jnp.float32)], axis=1)
    cnt_ref[...] += lax.dot_general(oh, jnp.ones((_BMF, HD), jnp.float32),
                                    (((1,), (0,)), ((), ())),
                                    preferred_element_type=jnp.float32)

    @pl.when(i == pl.num_programs(0) - 1)
    def _():
        pooled = acc_ref[...] / jnp.maximum(cnt_ref[:, :1], 1.0)
        o_ref[...] = lax.dot_general(pooled, wf_ref[...],
                                     (((1,), (1,)), ((), ())),
                                     preferred_element_type=jnp.float32
                                     ) + bf_ref[...]


def _tc_fin(agg, b, batch3, Wf, bf):
    nbf = NP // _BMF
    return pl.pallas_call(
        _tc_fin_body,
        grid=(nbf,),
        in_specs=[
            pl.BlockSpec((_BMF, HD), lambda i: (i, 0)),
            pl.BlockSpec((_BMF, HD), lambda i, nbf=nbf: (i + nbf, 0)),
            pl.BlockSpec((1, DH), lambda i: (0, 0)),
            pl.BlockSpec((1, 1, _BMF), lambda i: (i, 0, 0)),
            pl.BlockSpec((DOUT, DH), lambda i: (0, 0)),
            pl.BlockSpec((1, DOUT), lambda i: (0, 0)),
        ],
        out_specs=pl.BlockSpec((NB, DOUT), lambda i: (0, 0)),
        out_shape=jax.ShapeDtypeStruct((NB, DOUT), jnp.float32),
        scratch_shapes=[
            pltpu.VMEM((NB, DH), jnp.float32),
            pltpu.VMEM((NB, HD), jnp.float32),
        ],
    )(agg, agg, b, batch3, Wf, bf)


@functools.partial(
    pl.kernel,
    out_type=jax.ShapeDtypeStruct((EROWS, CE), jnp.float32),
    mesh=_mesh,
    compiler_params=_scp,
    scratch_types=[
        pltpu.VMEM((NP, 2), jnp.float32),
        pltpu.VMEM((NBLK, CE), jnp.int32),
        pltpu.VMEM((NBLK, CE), jnp.int32),
        pltpu.VMEM((NBLK, CE), jnp.float32),
        pltpu.VMEM((SROWS, CE), jnp.float32),
        pltpu.VMEM((16,), jnp.float32),
        pltpu.VMEM((16, 16), jnp.float32),
        pltpu.VMEM((SROWS,), jnp.int32),
        pltpu.VMEM((NBLK,), jnp.int32),
        pltpu.SemaphoreType.DMA,
        pltpu.VMEM_SHARED((16, 16), jnp.float32),
        pltpu.VMEM_SHARED((SROWS, CE), jnp.float32),
    ],
)
def _sc_softmax(sc_hbm, src_hbm, dst_hbm, alpha_hbm,
                sc_v, src_v, dst_v, e_v, s_v, mx_v, mx16_v, rid_v, crid_v,
                sem, mx_sh, s_sh):
    sid = lax.axis_index("s")
    cid = lax.axis_index("c")
    row0 = sid * NBLK

    pltpu.sync_copy(sc_hbm, sc_v)

    def _cr(i, _):
        crid_v[pl.ds(i * 16, 16)] = row0 + i * 16 + lax.iota(jnp.int32, 16)
        return 0
    lax.fori_loop(0, NBLK // 16, _cr, 0)
    pltpu.async_copy(src_hbm.at[crid_v], src_v, sem).wait()
    pltpu.async_copy(dst_hbm.at[crid_v], dst_v, sem).wait()

    def _z(i, _):
        s_v[i // 8, pl.ds((i % 8) * 16, 16)] = jnp.zeros((16,), jnp.float32)
        return 0
    lax.fori_loop(0, SROWS * 8, _z, 0)

    def _r(i, _):
        rid_v[pl.ds(i * 16, 16)] = i * 16 + lax.iota(jnp.int32, 16)
        return 0
    lax.fori_loop(0, SROWS // 16, _r, 0)

    @pl.when(sid == 0)
    def _():
        pltpu.sync_copy(s_v, s_sh)

    ebase = sid * T_E

    def _e(i, vmax):
        j = i // 8
        k = i % 8
        sl = pl.ds(k * 16, 16)
        isrc = src_v[j, sl]
        idst = dst_v[j, sl]
        zi = jnp.zeros((16,), jnp.int32)
        e = (plsc.load_gather(sc_v, [isrc, zi])
             + plsc.load_gather(sc_v, [idst, zi + 1]))
        e = jnp.where(e >= 0.0, e, e * 0.2)
        gid = ebase + j * CE + k * 16 + lax.iota(jnp.int32, 16)
        e = jnp.where(gid < E_TOT, e, NEG)
        e_v[j, sl] = e
        return jnp.maximum(vmax, e)

    vmax = lax.fori_loop(0, NBLK * 8, _e, jnp.full((16,), NEG, jnp.float32))
    mx_v[...] = vmax
    pltpu.sync_copy(mx_v, mx_sh.at[sid])
    plsc.subcore_barrier()

    pltpu.sync_copy(mx_sh, mx16_v)

    def _m(i, vm):
        return jnp.maximum(vm, mx16_v[i])
    g = jnp.max(lax.fori_loop(0, 16, _m, jnp.full((16,), NEG, jnp.float32)))

    def _p(i, _):
        j = i // 8
        k = i % 8
        sl = pl.ds(k * 16, 16)
        p = jnp.exp(e_v[j, sl] - g)
        e_v[j, sl] = p
        idst = dst_v[j, sl]
        plsc.addupdate_scatter(s_v, [idst >> 7, idst & 127], p)
        return 0
    lax.fori_loop(0, NBLK * 8, _p, 0)

    pltpu.sync_copy(s_v, s_sh.at[rid_v], add=True)
    plsc.subcore_barrier()
    pltpu.sync_copy(s_sh, s_v)

    def _a(i, _):
        j = i // 8
        k = i % 8
        sl = pl.ds(k * 16, 16)
        idst = dst_v[j, sl]
        s = plsc.load_gather(s_v, [idst >> 7, idst & 127])
        e_v[j, sl] = e_v[j, sl] / (s + 1e-16)
        return 0
    lax.fori_loop(0, NBLK * 8, _a, 0)

    half = NBLK // 2
    pltpu.sync_copy(e_v.at[pl.ds(cid * half, half)],
                    alpha_hbm.at[pl.ds(row0 + cid * half, half)])


_ZR = ACC_R // 16


@functools.partial(
    pl.kernel,
    out_type=jax.ShapeDtypeStruct((2 * NP, HD), jnp.float32),
    mesh=_mesh,
    compiler_params=_scp,
    scratch_types=[
        pltpu.VMEM((NBLK,), jnp.int32),
        pltpu.VMEM((NBLK, CE), jnp.int32),
        pltpu.VMEM((NBLK, CE), jnp.int32),
        pltpu.VMEM((NBLK, CE), jnp.int32),
        pltpu.VMEM((NBLK, CE), jnp.float32),
        pltpu.VMEM((CE, HD), jnp.float32),
        pltpu.VMEM((41, HD), jnp.float32),
        pltpu.SemaphoreType.DMA,
        pltpu.VMEM_SHARED((ACC_R, HD), jnp.float32),
    ],
)
def _sc_agg(hh_hbm, src_hbm, dst_hbm, alpha_hbm, agg_hbm,
            rid_v, src_v, dst_v, dr_v, al_v, g_v, z_v, sem, acc_sh):
    sid = lax.axis_index("s")
    cid = lax.axis_index("c")
    row0 = sid * NBLK

    def _ri(i, _):
        rid_v[pl.ds(i * 16, 16)] = row0 + i * 16 + lax.iota(jnp.int32, 16)
        return 0
    lax.fori_loop(0, NBLK // 16, _ri, 0)
    pltpu.async_copy(src_hbm.at[rid_v], src_v, sem).wait()
    pltpu.async_copy(dst_hbm.at[rid_v], dst_v, sem).wait()
    pltpu.async_copy(alpha_hbm.at[rid_v], al_v, sem).wait()

    hoff = cid * NP

    def _sh(i, _):
        j = i // 8
        sl = pl.ds((i % 8) * 16, 16)
        src_v[j, sl] = src_v[j, sl] + hoff
        return 0
    lax.fori_loop(0, NBLK * 8, _sh, 0)

    def _z(i, _):
        z_v[i // 8, pl.ds((i % 8) * 16, 16)] = jnp.zeros((16,), jnp.float32)
        return 0
    lax.fori_loop(0, 41 * 8, _z, 0)

    def _pass(p, _):
        base = p * PASS_R

        def _zc(t, _):
            pltpu.sync_copy(z_v, acc_sh.at[pl.ds(sid * _ZR + t * 41, 41)])
            return 0
        lax.fori_loop(0, 8, _zc, 0)

        def _rm(i, _):
            j = i // 8
            sl = pl.ds((i % 8) * 16, 16)
            d = dst_v[j, sl] - base
            ok = (d >= 0) & (d < PASS_R)
            dr_v[j, sl] = jnp.where(ok, d, ACC_R - 1)
            return 0
        lax.fori_loop(0, NBLK * 8, _rm, 0)
        plsc.subcore_barrier()

        def _blk(j, _):
            pltpu.async_copy(hh_hbm.at[src_v.at[j]], g_v, sem).wait()

            def _row(r, _):
                zi = jnp.zeros((16,), jnp.int32)
                a = plsc.load_gather(al_v, [zi + j, zi + r])
                for k in range(8):
                    sl = pl.ds(k * 16, 16)
                    g_v[r, sl] = g_v[r, sl] * a
                return 0
            lax.fori_loop(0, CE, _row, 0)

            pltpu.sync_copy(g_v, acc_sh.at[dr_v.at[j]], add=True)
            return 0

        lax.fori_loop(0, NBLK, _blk, 0)
        plsc.subcore_barrier()

        pltpu.sync_copy(
            acc_sh.at[pl.ds(sid * (PASS_R // 16), PASS_R // 16)],
            agg_hbm.at[pl.ds(cid * NP + base + sid * (PASS_R // 16),
                             PASS_R // 16)])
        plsc.subcore_barrier()
        return 0

    lax.fori_loop(0, NPASS, _pass, 0)


def _layer(hh, sc, src2, dst2):
    alpha = _sc_softmax(sc, src2, dst2)
    return _sc_agg(hh, src2, dst2, alpha)


def kernel(x, edge_index, batch, W1, att_src1, att_dst1, b1,
           W2, att_src2, att_dst2, b2, W3, att_src3, att_dst3, b3, Wf, bf):
    loop = jnp.arange(N, dtype=jnp.int32)
    pad = jnp.zeros((E_PAD - E_TOT,), jnp.int32)
    src2 = jnp.concatenate([edge_index[0].astype(jnp.int32), loop, pad]
                           ).reshape(EROWS, CE)
    dst2 = jnp.concatenate([edge_index[1].astype(jnp.int32), loop, pad]
                           ).reshape(EROWS, CE)
    batch3 = jnp.concatenate(
        [batch.astype(jnp.int32), jnp.full((NP - N,), NB, jnp.int32)]
    ).reshape(NP // _BMF, 1, _BMF)

    hh3, sc = _tc_in(x, W1, att_src1.reshape(1, DH), att_dst1.reshape(1, DH))
    agg = _layer(hh3.reshape(2 * NP, HD), sc, src2, dst2)
    hh3, sc = _tc_mid(agg, b1.reshape(1, DH), W2,
                      att_src2.reshape(1, DH), att_dst2.reshape(1, DH))
    agg = _layer(hh3.reshape(2 * NP, HD), sc, src2, dst2)
    hh3, sc = _tc_mid(agg, b2.reshape(1, DH), W3,
                      att_src3.reshape(1, DH), att_dst3.reshape(1, DH))
    agg = _layer(hh3.reshape(2 * NP, HD), sc, src2, dst2)
    return _tc_fin(agg, b3.reshape(1, DH), batch3, Wf, bf.reshape(1, DOUT))
